# baseline (speedup 1.0000x reference)
import functools

import jax
import jax.numpy as jnp
from jax import lax
from jax.experimental import pallas as pl
from jax.experimental.pallas import tpu as pltpu
from jax.experimental.pallas import tpu_sc as plsc

N = 10000
E = 320000
R = 8
D = 128

NC = 2
NS = 16
LANES = 128
ROWS_PER_TILE = 80
EP = NC * NS * ROWS_PER_TILE * LANES
EROWS = EP // LANES

NPAD = 10112
ACC_TILE_ROWS = NPAD // NS
CNT_BINS = N * R + 8
CNT_PAD = ((CNT_BINS + NS * 128 - 1) // (NS * 128)) * (NS * 128)
CNT_TILE = CNT_PAD // NS

_MESH = plsc.VectorSubcoreMesh(core_axis_name="c", subcore_axis_name="s")


def _sc_wid():
    c = lax.axis_index("c")
    s = lax.axis_index("s")
    return c, s, c * NS + s


def _sc_cnt_w_body(seg_hbm, zeros_hbm, w_hbm, cnt_sh, seg_v, ones_v, vals_v, w_v):
    c, s, wid = _sc_wid()

    def fill_ones(g, _):
        ones_v[pl.ds(g * 16, 16)] = jnp.ones((16,), jnp.float32)
        return 0
    lax.fori_loop(0, LANES // 16, fill_ones, 0)

    pltpu.sync_copy(zeros_hbm, cnt_sh.at[pl.ds(s * CNT_TILE, CNT_TILE)])
    plsc.subcore_barrier()

    rows_per_tile_a = EROWS // NS
    pltpu.sync_copy(seg_hbm.at[pl.ds(s * rows_per_tile_a, rows_per_tile_a)],
                    seg_v.at[pl.ds(0, rows_per_tile_a)])

    def hist_step(j, _):
        pltpu.sync_copy(ones_v, cnt_sh.at[seg_v.at[j]], add=True)
        return 0
    lax.fori_loop(0, rows_per_tile_a, hist_step, 0)
    plsc.subcore_barrier()

    base = wid * ROWS_PER_TILE
    pltpu.sync_copy(seg_hbm.at[pl.ds(base, ROWS_PER_TILE)],
                    seg_v.at[pl.ds(0, ROWS_PER_TILE)])

    def w_step(j, _):
        pltpu.sync_copy(cnt_sh.at[seg_v.at[j]], vals_v)
        def grp(g, _):
            cv = vals_v[pl.ds(g * 16, 16)]
            w_v[j, pl.ds(g * 16, 16)] = 1.0 / jnp.maximum(cv, 1.0)
            return 0
        lax.fori_loop(0, LANES // 16, grp, 0)
        return 0
    lax.fori_loop(0, ROWS_PER_TILE, w_step, 0)
    pltpu.sync_copy(w_v, w_hbm.at[pl.ds(base, ROWS_PER_TILE)])


def _sc_cnt_w(seg2d, zeros_cnt):
    return pl.kernel(
        _sc_cnt_w_body,
        out_type=jax.ShapeDtypeStruct((EROWS, LANES), jnp.float32),
        mesh=_MESH,
        scratch_types=[
            pltpu.VMEM_SHARED((CNT_PAD,), jnp.float32),
            pltpu.VMEM((EROWS // NS, LANES), jnp.int32),
            pltpu.VMEM((LANES,), jnp.float32),
            pltpu.VMEM((LANES,), jnp.float32),
            pltpu.VMEM((ROWS_PER_TILE, LANES), jnp.float32),
        ],
        name="sc_hist_w",
    )(seg2d, zeros_cnt)


KCH = 8
FAST_CORE = 0
RPT_FAST = 160
RPT_SLOW = 0


def _scale_chunk(rowbuf, w_v, j):
    def grp(g, _):
        w16 = w_v[j, pl.ds(g * 16, 16)]
        for l in range(16):
            ws = w16[l]
            e = g * 16 + l
            for v in range(D // 16):
                rowbuf[e, pl.ds(v * 16, 16)] = rowbuf[e, pl.ds(v * 16, 16)] * ws
        return 0
    lax.fori_loop(0, LANES // 16, grp, 0)


def _sc_agg_body(y_hbm, gidx_hbm, dst_hbm, w_hbm, zeros_hbm, p_hbm,
                 acc_sh, gidxs, dsts, ws, rows, seed_idx, gsems, ssems, isem):
    c, s, wid = _sc_wid()

    pltpu.sync_copy(zeros_hbm, acc_sh.at[pl.ds(s * ACC_TILE_ROWS, ACC_TILE_ROWS)])
    plsc.subcore_barrier()

    on_fast = c == FAST_CORE
    base = jnp.where(on_fast, s * RPT_FAST, NS * RPT_FAST + s * RPT_SLOW)
    base = jnp.minimum(base, EROWS - KCH)
    nblk_last = jnp.where(on_fast, RPT_FAST // KCH - 1, RPT_SLOW // KCH - 1)
    npairs = jnp.where(on_fast, RPT_FAST // (2 * KCH), RPT_SLOW // (2 * KCH))

    def load_idx_block(blk, q, sync):
        rs = pl.ds(pl.multiple_of(base + blk * KCH, 8), KCH)
        if sync:
            pltpu.sync_copy(gidx_hbm.at[rs], gidxs[q])
            pltpu.sync_copy(dst_hbm.at[rs], dsts[q])
            pltpu.sync_copy(w_hbm.at[rs], ws[q])
        else:
            pltpu.async_copy(gidx_hbm.at[rs], gidxs[q], isem)
            pltpu.async_copy(dst_hbm.at[rs], dsts[q], isem)
            pltpu.async_copy(w_hbm.at[rs], ws[q], isem)

    def wait_idx_block(q):
        pltpu.make_async_copy(gidx_hbm.at[pl.ds(0, KCH)], gidxs[q], isem).wait()
        pltpu.make_async_copy(dst_hbm.at[pl.ds(0, KCH)], dsts[q], isem).wait()
        pltpu.make_async_copy(w_hbm.at[pl.ds(0, KCH)], ws[q], isem).wait()

    def wait_gather(p):
        pltpu.make_async_copy(y_hbm.at[pl.ds(0, LANES)], rows[p], gsems[p]).wait()

    def wait_scatter(p):
        pltpu.make_async_copy(rows[p], acc_sh.at[pl.ds(0, LANES)], ssems[p]).wait()

    load_idx_block(0, 0, sync=True)
    for g in range(LANES // 16):
        seed_idx[0, pl.ds(g * 16, 16)] = jnp.full((16,), N, jnp.int32)
    for p in range(2):
        pltpu.sync_copy(zeros_hbm.at[pl.ds(0, LANES)], rows[p])
        pltpu.async_copy(rows[p], acc_sh.at[seed_idx.at[0]], ssems[p], add=True)
    wait_scatter(0)
    pltpu.async_copy(y_hbm.at[gidxs[0].at[0]], rows[0], gsems[0])

    def blk_pair(k2, _):
        for q in range(2):
            blk = k2 * 2 + q
            nblk = jnp.minimum(blk + 1, nblk_last)
            load_idx_block(nblk, 1 - q, sync=False)

            def chunk_pair(j2, _):
                for p in range(2):
                    j = j2 * 2 + p
                    wait_gather(p)
                    _scale_chunk(rows[p], ws[q], j)
                    wait_scatter(1 - p)
                    pltpu.async_copy(y_hbm.at[gidxs[q].at[j + 1]],
                                     rows[1 - p], gsems[1 - p])
                    pltpu.async_copy(rows[p], acc_sh.at[dsts[q].at[j]],
                                     ssems[p], add=True)
                return 0
            lax.fori_loop(0, KCH // 2 - 1, chunk_pair, 0)

            wait_gather(0)
            _scale_chunk(rows[0], ws[q], KCH - 2)
            wait_scatter(1)
            pltpu.async_copy(y_hbm.at[gidxs[q].at[KCH - 1]], rows[1], gsems[1])
            pltpu.async_copy(rows[0], acc_sh.at[dsts[q].at[KCH - 2]],
                             ssems[0], add=True)
            wait_idx_block(1 - q)
            wait_gather(1)
            _scale_chunk(rows[1], ws[q], KCH - 1)
            wait_scatter(0)
            pltpu.async_copy(y_hbm.at[gidxs[1 - q].at[0]], rows[0], gsems[0])
            pltpu.async_copy(rows[1], acc_sh.at[dsts[q].at[KCH - 1]],
                             ssems[1], add=True)
        return 0
    lax.fori_loop(0, npairs, blk_pair, 0)

    wait_gather(0)
    wait_scatter(1)
    plsc.subcore_barrier()

    out_base = c * NPAD + s * ACC_TILE_ROWS
    pltpu.sync_copy(acc_sh.at[pl.ds(s * ACC_TILE_ROWS, ACC_TILE_ROWS)],
                    p_hbm.at[pl.ds(out_base, ACC_TILE_ROWS)])


def _sc_agg(y2d, gidx2d, dst2d, w2d, zeros_acc):
    return pl.kernel(
        _sc_agg_body,
        out_type=jax.ShapeDtypeStruct((NC * NPAD, D), jnp.float32),
        mesh=_MESH,
        scratch_types=[
            pltpu.VMEM_SHARED((NPAD, D), jnp.float32),
            [pltpu.VMEM((KCH, LANES), jnp.int32) for _ in range(2)],
            [pltpu.VMEM((KCH, LANES), jnp.int32) for _ in range(2)],
            [pltpu.VMEM((KCH, LANES), jnp.float32) for _ in range(2)],
            [pltpu.VMEM((LANES, D), jnp.float32) for _ in range(2)],
            pltpu.VMEM((1, LANES), jnp.int32),
            [pltpu.SemaphoreType.DMA for _ in range(2)],
            [pltpu.SemaphoreType.DMA for _ in range(2)],
            pltpu.SemaphoreType.DMA,
        ],
        name="sc_gather_scale_scatter",
    )(y2d, gidx2d, dst2d, w2d, zeros_acc)


BN = 1000


def _tc_y_body(x_ref, w_ref, y_ref):
    x = x_ref[...]
    for r in range(R):
        y_ref[r] = jnp.dot(x, w_ref[r], preferred_element_type=jnp.float32)


def _tc_y(X, W):
    return pl.pallas_call(
        _tc_y_body,
        grid=(N // BN,),
        in_specs=[
            pl.BlockSpec((BN, D), lambda i: (i, 0)),
            pl.BlockSpec((R, D, D), lambda i: (0, 0, 0)),
        ],
        out_specs=pl.BlockSpec((R, BN, D), lambda i: (0, i, 0)),
        out_shape=jax.ShapeDtypeStruct((R, N, D), jnp.float32),
    )(X, W)


def _tc_mid_body(x_ref, ws_ref, p0_ref, p1_ref, w2_ref, h_ref, y_ref):
    x = x_ref[...]
    h = p0_ref[...] + p1_ref[...] + jnp.dot(x, ws_ref[...],
                                            preferred_element_type=jnp.float32)
    h = jnp.maximum(h, 0.0)
    h_ref[...] = h
    for r in range(R):
        y_ref[r] = jnp.dot(h, w2_ref[r], preferred_element_type=jnp.float32)


def _tc_mid(X, Wself1, P0, P1, W2):
    return pl.pallas_call(
        _tc_mid_body,
        grid=(N // BN,),
        in_specs=[
            pl.BlockSpec((BN, D), lambda i: (i, 0)),
            pl.BlockSpec((D, D), lambda i: (0, 0)),
            pl.BlockSpec((BN, D), lambda i: (i, 0)),
            pl.BlockSpec((BN, D), lambda i: (i, 0)),
            pl.BlockSpec((R, D, D), lambda i: (0, 0, 0)),
        ],
        out_specs=[
            pl.BlockSpec((BN, D), lambda i: (i, 0)),
            pl.BlockSpec((R, BN, D), lambda i: (0, i, 0)),
        ],
        out_shape=[
            jax.ShapeDtypeStruct((N, D), jnp.float32),
            jax.ShapeDtypeStruct((R, N, D), jnp.float32),
        ],
    )(X, Wself1, P0, P1, W2)


def _tc_final_body(h_ref, ws_ref, p0_ref, p1_ref, o_ref):
    o_ref[...] = p0_ref[...] + p1_ref[...] + jnp.dot(
        h_ref[...], ws_ref[...], preferred_element_type=jnp.float32)


def _tc_final(h, Wself2, P0, P1):
    return pl.pallas_call(
        _tc_final_body,
        grid=(N // BN,),
        in_specs=[
            pl.BlockSpec((BN, D), lambda i: (i, 0)),
            pl.BlockSpec((D, D), lambda i: (0, 0)),
            pl.BlockSpec((BN, D), lambda i: (i, 0)),
            pl.BlockSpec((BN, D), lambda i: (i, 0)),
        ],
        out_specs=pl.BlockSpec((BN, D), lambda i: (i, 0)),
        out_shape=jax.ShapeDtypeStruct((N, D), jnp.float32),
    )(h, Wself2, P0, P1)


def kernel(X, edge_index, edge_type, W1, Wself1, W2, Wself2, epoch):
    src = edge_index[0]
    dst = edge_index[1]
    et = edge_type

    pad = EP - E
    gidx = jnp.pad(et * N + src, (0, pad)).reshape(EROWS, LANES)
    dstp = jnp.pad(dst, (0, pad), constant_values=N).reshape(EROWS, LANES)
    seg = jnp.pad(dst * R + et, (0, pad), constant_values=N * R).reshape(EROWS, LANES)

    zeros_cnt = jnp.zeros((CNT_TILE,), jnp.float32)
    zeros_acc = jnp.zeros((ACC_TILE_ROWS, D), jnp.float32)

    w2d = _sc_cnt_w(seg, zeros_cnt)

    Y1 = _tc_y(X, W1).reshape(R * N, D)
    P = _sc_agg(Y1, gidx, dstp, w2d, zeros_acc)
    P0 = lax.slice(P, (0, 0), (N, D))
    P1 = lax.slice(P, (NPAD, 0), (NPAD + N, D))

    h, Y2 = _tc_mid(X, Wself1, P0, P1, W2)
    Y2 = Y2.reshape(R * N, D)
    Q = _sc_agg(Y2, gidx, dstp, w2d, zeros_acc)
    Q0 = lax.slice(Q, (0, 0), (N, D))
    Q1 = lax.slice(Q, (NPAD, 0), (NPAD + N, D))

    return _tc_final(h, Wself2, Q0, Q1)


# speedup vs baseline: 1.9089x; 1.9089x over previous
import functools

import jax
import jax.numpy as jnp
from jax import lax
from jax.experimental import pallas as pl
from jax.experimental.pallas import tpu as pltpu
from jax.experimental.pallas import tpu_sc as plsc

N = 10000
E = 320000
R = 8
D = 128

NC = 2
NS = 16
LANES = 128
ROWS_PER_TILE = 80
EP = NC * NS * ROWS_PER_TILE * LANES
EROWS = EP // LANES

NPAD = 10112
ACC_TILE_ROWS = NPAD // NS
CNT_BINS = N * R + 8
CNT_PAD = ((CNT_BINS + NS * 128 - 1) // (NS * 128)) * (NS * 128)
CNT_TILE = CNT_PAD // NS

_MESH = plsc.VectorSubcoreMesh(core_axis_name="c", subcore_axis_name="s")


def _sc_wid():
    c = lax.axis_index("c")
    s = lax.axis_index("s")
    return c, s, c * NS + s


def _sc_cnt_w_body(seg_hbm, zeros_hbm, w_hbm, cnt_sh, seg_v, ones_v, vals_v, w_v):
    c, s, wid = _sc_wid()

    def fill_ones(g, _):
        ones_v[pl.ds(g * 16, 16)] = jnp.ones((16,), jnp.float32)
        return 0
    lax.fori_loop(0, LANES // 16, fill_ones, 0)

    pltpu.sync_copy(zeros_hbm, cnt_sh.at[pl.ds(s * CNT_TILE, CNT_TILE)])
    plsc.subcore_barrier()

    rows_per_tile_a = EROWS // NS
    pltpu.sync_copy(seg_hbm.at[pl.ds(s * rows_per_tile_a, rows_per_tile_a)],
                    seg_v.at[pl.ds(0, rows_per_tile_a)])

    def hist_step(j, _):
        pltpu.sync_copy(ones_v, cnt_sh.at[seg_v.at[j]], add=True)
        return 0
    lax.fori_loop(0, rows_per_tile_a, hist_step, 0)
    plsc.subcore_barrier()

    base = wid * ROWS_PER_TILE
    pltpu.sync_copy(seg_hbm.at[pl.ds(base, ROWS_PER_TILE)],
                    seg_v.at[pl.ds(0, ROWS_PER_TILE)])

    def w_step(j, _):
        pltpu.sync_copy(cnt_sh.at[seg_v.at[j]], vals_v)
        def grp(g, _):
            cv = vals_v[pl.ds(g * 16, 16)]
            w_v[j, pl.ds(g * 16, 16)] = 1.0 / jnp.maximum(cv, 1.0)
            return 0
        lax.fori_loop(0, LANES // 16, grp, 0)
        return 0
    lax.fori_loop(0, ROWS_PER_TILE, w_step, 0)
    pltpu.sync_copy(w_v, w_hbm.at[pl.ds(base, ROWS_PER_TILE)])


def _sc_cnt_w(seg2d, zeros_cnt):
    return pl.kernel(
        _sc_cnt_w_body,
        out_type=jax.ShapeDtypeStruct((EROWS, LANES), jnp.float32),
        mesh=_MESH,
        scratch_types=[
            pltpu.VMEM_SHARED((CNT_PAD,), jnp.float32),
            pltpu.VMEM((EROWS // NS, LANES), jnp.int32),
            pltpu.VMEM((LANES,), jnp.float32),
            pltpu.VMEM((LANES,), jnp.float32),
            pltpu.VMEM((ROWS_PER_TILE, LANES), jnp.float32),
        ],
        name="sc_hist_w",
    )(seg2d, zeros_cnt)


KCH = 8
FAST_CORE = 0
RPT_FAST = 144
RPT_SLOW = 16


def _scale_chunk(rowbuf, w_v, j):
    def grp(g, _):
        w16 = w_v[j, pl.ds(g * 16, 16)]
        for l in range(16):
            ws = w16[l]
            e = g * 16 + l
            for v in range(D // 16):
                rowbuf[e, pl.ds(v * 16, 16)] = rowbuf[e, pl.ds(v * 16, 16)] * ws
        return 0
    lax.fori_loop(0, LANES // 16, grp, 0)


def _sc_agg_body(y_hbm, gidx_hbm, dst_hbm, w_hbm, zeros_hbm, p_hbm,
                 acc_sh, gidxs, dsts, ws, rows, seed_idx, gsems, ssems, isem):
    c, s, wid = _sc_wid()

    pltpu.sync_copy(zeros_hbm, acc_sh.at[pl.ds(s * ACC_TILE_ROWS, ACC_TILE_ROWS)])
    plsc.subcore_barrier()

    on_fast = c == FAST_CORE
    base = jnp.where(on_fast, s * RPT_FAST, NS * RPT_FAST + s * RPT_SLOW)
    base = jnp.minimum(base, EROWS - KCH)
    nblk_last = jnp.where(on_fast, RPT_FAST // KCH - 1, RPT_SLOW // KCH - 1)
    npairs = jnp.where(on_fast, RPT_FAST // (2 * KCH), RPT_SLOW // (2 * KCH))

    def load_idx_block(blk, q, sync):
        rs = pl.ds(pl.multiple_of(base + blk * KCH, 8), KCH)
        if sync:
            pltpu.sync_copy(gidx_hbm.at[rs], gidxs[q])
            pltpu.sync_copy(dst_hbm.at[rs], dsts[q])
            pltpu.sync_copy(w_hbm.at[rs], ws[q])
        else:
            pltpu.async_copy(gidx_hbm.at[rs], gidxs[q], isem)
            pltpu.async_copy(dst_hbm.at[rs], dsts[q], isem)
            pltpu.async_copy(w_hbm.at[rs], ws[q], isem)

    def wait_idx_block(q):
        pltpu.make_async_copy(gidx_hbm.at[pl.ds(0, KCH)], gidxs[q], isem).wait()
        pltpu.make_async_copy(dst_hbm.at[pl.ds(0, KCH)], dsts[q], isem).wait()
        pltpu.make_async_copy(w_hbm.at[pl.ds(0, KCH)], ws[q], isem).wait()

    def wait_gather(p):
        pltpu.make_async_copy(y_hbm.at[pl.ds(0, LANES)], rows[p], gsems[p]).wait()

    def wait_scatter(p):
        pltpu.make_async_copy(rows[p], acc_sh.at[pl.ds(0, LANES)], ssems[p]).wait()

    load_idx_block(0, 0, sync=True)
    for g in range(LANES // 16):
        seed_idx[0, pl.ds(g * 16, 16)] = jnp.full((16,), N, jnp.int32)
    for p in range(2):
        pltpu.sync_copy(zeros_hbm.at[pl.ds(0, LANES)], rows[p])
        pltpu.async_copy(rows[p], acc_sh.at[seed_idx.at[0]], ssems[p], add=True)
    wait_scatter(0)
    pltpu.async_copy(y_hbm.at[gidxs[0].at[0]], rows[0], gsems[0])

    def blk_pair(k2, _):
        for q in range(2):
            blk = k2 * 2 + q
            nblk = jnp.minimum(blk + 1, nblk_last)
            load_idx_block(nblk, 1 - q, sync=False)

            def chunk_pair(j2, _):
                for p in range(2):
                    j = j2 * 2 + p
                    wait_gather(p)
                    _scale_chunk(rows[p], ws[q], j)
                    wait_scatter(1 - p)
                    pltpu.async_copy(y_hbm.at[gidxs[q].at[j + 1]],
                                     rows[1 - p], gsems[1 - p])
                    pltpu.async_copy(rows[p], acc_sh.at[dsts[q].at[j]],
                                     ssems[p], add=True)
                return 0
            lax.fori_loop(0, KCH // 2 - 1, chunk_pair, 0)

            wait_gather(0)
            _scale_chunk(rows[0], ws[q], KCH - 2)
            wait_scatter(1)
            pltpu.async_copy(y_hbm.at[gidxs[q].at[KCH - 1]], rows[1], gsems[1])
            pltpu.async_copy(rows[0], acc_sh.at[dsts[q].at[KCH - 2]],
                             ssems[0], add=True)
            wait_idx_block(1 - q)
            wait_gather(1)
            _scale_chunk(rows[1], ws[q], KCH - 1)
            wait_scatter(0)
            pltpu.async_copy(y_hbm.at[gidxs[1 - q].at[0]], rows[0], gsems[0])
            pltpu.async_copy(rows[1], acc_sh.at[dsts[q].at[KCH - 1]],
                             ssems[1], add=True)
        return 0
    lax.fori_loop(0, npairs, blk_pair, 0)

    wait_gather(0)
    wait_scatter(1)
    plsc.subcore_barrier()

    out_base = c * NPAD + s * ACC_TILE_ROWS
    pltpu.sync_copy(acc_sh.at[pl.ds(s * ACC_TILE_ROWS, ACC_TILE_ROWS)],
                    p_hbm.at[pl.ds(out_base, ACC_TILE_ROWS)])


def _sc_agg(y2d, gidx2d, dst2d, w2d, zeros_acc):
    return pl.kernel(
        _sc_agg_body,
        out_type=jax.ShapeDtypeStruct((NC * NPAD, D), jnp.float32),
        mesh=_MESH,
        scratch_types=[
            pltpu.VMEM_SHARED((NPAD, D), jnp.float32),
            [pltpu.VMEM((KCH, LANES), jnp.int32) for _ in range(2)],
            [pltpu.VMEM((KCH, LANES), jnp.int32) for _ in range(2)],
            [pltpu.VMEM((KCH, LANES), jnp.float32) for _ in range(2)],
            [pltpu.VMEM((LANES, D), jnp.float32) for _ in range(2)],
            pltpu.VMEM((1, LANES), jnp.int32),
            [pltpu.SemaphoreType.DMA for _ in range(2)],
            [pltpu.SemaphoreType.DMA for _ in range(2)],
            pltpu.SemaphoreType.DMA,
        ],
        name="sc_gather_scale_scatter",
    )(y2d, gidx2d, dst2d, w2d, zeros_acc)


BN = 1000


def _tc_y_body(x_ref, w_ref, y_ref):
    x = x_ref[...]
    for r in range(R):
        y_ref[r] = jnp.dot(x, w_ref[r], preferred_element_type=jnp.float32)


def _tc_y(X, W):
    return pl.pallas_call(
        _tc_y_body,
        grid=(N // BN,),
        in_specs=[
            pl.BlockSpec((BN, D), lambda i: (i, 0)),
            pl.BlockSpec((R, D, D), lambda i: (0, 0, 0)),
        ],
        out_specs=pl.BlockSpec((R, BN, D), lambda i: (0, i, 0)),
        out_shape=jax.ShapeDtypeStruct((R, N, D), jnp.float32),
    )(X, W)


def _tc_mid_body(x_ref, ws_ref, p0_ref, p1_ref, w2_ref, h_ref, y_ref):
    x = x_ref[...]
    h = p0_ref[...] + p1_ref[...] + jnp.dot(x, ws_ref[...],
                                            preferred_element_type=jnp.float32)
    h = jnp.maximum(h, 0.0)
    h_ref[...] = h
    for r in range(R):
        y_ref[r] = jnp.dot(h, w2_ref[r], preferred_element_type=jnp.float32)


def _tc_mid(X, Wself1, P0, P1, W2):
    return pl.pallas_call(
        _tc_mid_body,
        grid=(N // BN,),
        in_specs=[
            pl.BlockSpec((BN, D), lambda i: (i, 0)),
            pl.BlockSpec((D, D), lambda i: (0, 0)),
            pl.BlockSpec((BN, D), lambda i: (i, 0)),
            pl.BlockSpec((BN, D), lambda i: (i, 0)),
            pl.BlockSpec((R, D, D), lambda i: (0, 0, 0)),
        ],
        out_specs=[
            pl.BlockSpec((BN, D), lambda i: (i, 0)),
            pl.BlockSpec((R, BN, D), lambda i: (0, i, 0)),
        ],
        out_shape=[
            jax.ShapeDtypeStruct((N, D), jnp.float32),
            jax.ShapeDtypeStruct((R, N, D), jnp.float32),
        ],
    )(X, Wself1, P0, P1, W2)


def _tc_final_body(h_ref, ws_ref, p0_ref, p1_ref, o_ref):
    o_ref[...] = p0_ref[...] + p1_ref[...] + jnp.dot(
        h_ref[...], ws_ref[...], preferred_element_type=jnp.float32)


def _tc_final(h, Wself2, P0, P1):
    return pl.pallas_call(
        _tc_final_body,
        grid=(N // BN,),
        in_specs=[
            pl.BlockSpec((BN, D), lambda i: (i, 0)),
            pl.BlockSpec((D, D), lambda i: (0, 0)),
            pl.BlockSpec((BN, D), lambda i: (i, 0)),
            pl.BlockSpec((BN, D), lambda i: (i, 0)),
        ],
        out_specs=pl.BlockSpec((BN, D), lambda i: (i, 0)),
        out_shape=jax.ShapeDtypeStruct((N, D), jnp.float32),
    )(h, Wself2, P0, P1)


def kernel(X, edge_index, edge_type, W1, Wself1, W2, Wself2, epoch):
    src = edge_index[0]
    dst = edge_index[1]
    et = edge_type

    pad = EP - E
    gidx = jnp.pad(et * N + src, (0, pad)).reshape(EROWS, LANES)
    dstp = jnp.pad(dst, (0, pad), constant_values=N).reshape(EROWS, LANES)
    seg = jnp.pad(dst * R + et, (0, pad), constant_values=N * R).reshape(EROWS, LANES)

    zeros_cnt = jnp.zeros((CNT_TILE,), jnp.float32)
    zeros_acc = jnp.zeros((ACC_TILE_ROWS, D), jnp.float32)

    w2d = _sc_cnt_w(seg, zeros_cnt)

    Y1 = _tc_y(X, W1).reshape(R * N, D)
    P = _sc_agg(Y1, gidx, dstp, w2d, zeros_acc)
    P0 = lax.slice(P, (0, 0), (N, D))
    P1 = lax.slice(P, (NPAD, 0), (NPAD + N, D))

    h, Y2 = _tc_mid(X, Wself1, P0, P1, W2)
    Y2 = Y2.reshape(R * N, D)
    Q = _sc_agg(Y2, gidx, dstp, w2d, zeros_acc)
    Q0 = lax.slice(Q, (0, 0), (N, D))
    Q1 = lax.slice(Q, (NPAD, 0), (NPAD + N, D))

    return _tc_final(h, Wself2, Q0, Q1)
